# baseline (speedup 1.0000x reference)
import functools

import jax
import jax.numpy as jnp
from jax import lax
from jax.experimental import pallas as pl
from jax.experimental.pallas import tpu as pltpu
from jax.experimental.pallas import tpu_sc as plsc

_F = 26
_V = 100000
_D = 32
_B = 4096
_NC = 2
_NS = 16
_NW = _NC * _NS
_L = 16
_CHUNK = 128
_NSPLIT = 2
_FS = _F // _NSPLIT
_POS_S = _B * _FS
_POS_W = _POS_S // _NW
_NCHUNK = _POS_W // _CHUNK

_mesh = plsc.VectorSubcoreMesh(core_axis_name="c", subcore_axis_name="s")


def _make_split(s):
    @functools.partial(
        pl.kernel,
        mesh=_mesh,
        out_type=jax.ShapeDtypeStruct((_FS * _D, _B), jnp.float32),
        scratch_types=[
            pltpu.VMEM((_NCHUNK, _CHUNK), jnp.int32),
            pltpu.VMEM((_D, _POS_W), jnp.float32),
            pltpu.SemaphoreType.DMA,
        ],
        compiler_params=pltpu.CompilerParams(use_tc_tiling_on_sc=False),
        name=f"emb_gather_s{s}",
    )
    def _emb(tableS, idxT3, out, idx_v, val_v, sem):
        wid = lax.axis_index("s") * _NC + lax.axis_index("c")
        base = wid * _POS_W
        pltpu.sync_copy(idxT3.at[wid], idx_v)

        def _fire(j, _):
            fl = (base + j * _CHUNK) // _B
            for d in range(_D):
                pltpu.async_copy(
                    tableS.at[fl * _D + d].at[idx_v.at[j]],
                    val_v.at[d, pl.ds(j * _CHUNK, _CHUNK)],
                    sem,
                )
            return _

        lax.fori_loop(0, _NCHUNK, _fire, None)
        pltpu.make_async_copy(
            out.at[pl.ds(0, _D), pl.ds(0, _POS_W)], val_v, sem).wait()

        def _wb(j, _):
            p = base + j * _CHUNK
            fl = p // _B
            b = p - fl * _B
            pltpu.sync_copy(
                val_v.at[:, pl.ds(j * _CHUNK, _CHUNK)],
                out.at[pl.ds(fl * _D, _D), pl.ds(b, _CHUNK)],
            )
            return _

        lax.fori_loop(0, _NCHUNK, _wb, None)

    return _emb


_kernels = [_make_split(s) for s in range(_NSPLIT)]


def kernel(x, W):
    tableT = W.transpose(0, 2, 1).reshape(_F * _D, _V)
    idxT = x.astype(jnp.int32).T.reshape(_F, _B)
    outs = []
    for s in range(_NSPLIT):
        tableS = tableT[s * _FS * _D:(s + 1) * _FS * _D]
        idxS = idxT[s * _FS:(s + 1) * _FS].reshape(_NW, _NCHUNK, _CHUNK)
        outs.append(_kernels[s](tableS, idxS))
    outT = jnp.concatenate(outs, axis=0)
    return outT.reshape(_F, _D, _B).transpose(2, 0, 1)


# speedup vs baseline: 1.2972x; 1.2972x over previous
import functools

import jax
import jax.numpy as jnp
from jax import lax
from jax.experimental import pallas as pl
from jax.experimental.pallas import tpu as pltpu
from jax.experimental.pallas import tpu_sc as plsc

_F = 26
_V = 100000
_D = 32
_B = 4096
_NC = 2
_NS = 16
_NW = _NC * _NS
_L = 16
_CHUNK = 128
_POS_W = _B * _F // _NW
_NCHUNK = _POS_W // _CHUNK

_mesh = plsc.VectorSubcoreMesh(core_axis_name="c", subcore_axis_name="s")


@functools.partial(
    pl.kernel,
    mesh=_mesh,
    out_type=jax.ShapeDtypeStruct((_F, _D, _B), jnp.float32),
    scratch_types=[
        pltpu.VMEM((_NCHUNK, _CHUNK), jnp.int32),
        pltpu.VMEM((_D, _POS_W), jnp.float32),
        pltpu.SemaphoreType.DMA,
    ],
    compiler_params=pltpu.CompilerParams(use_tc_tiling_on_sc=False),
)
def _emb_t(tableT, idxT3, out, idx_v, val_v, sem):
    wid = lax.axis_index("s") * _NC + lax.axis_index("c")
    base = wid * _POS_W
    pltpu.sync_copy(idxT3.at[wid], idx_v)

    def _fire(j, _):
        f = (base + j * _CHUNK) // _B
        for d in range(_D):
            pltpu.async_copy(
                tableT.at[f, d].at[idx_v.at[j]],
                val_v.at[d, pl.ds(j * _CHUNK, _CHUNK)],
                sem,
            )
        return _

    lax.fori_loop(0, _NCHUNK, _fire, None)

    pltpu.make_async_copy(
        out.at[0, pl.ds(0, _D), pl.ds(0, _POS_W)], val_v, sem).wait()

    def _wb(j, _):
        p = base + j * _CHUNK
        f = p // _B
        b = p - f * _B
        pltpu.sync_copy(
            val_v.at[:, pl.ds(j * _CHUNK, _CHUNK)],
            out.at[f, :, pl.ds(b, _CHUNK)],
        )
        return _

    lax.fori_loop(0, _NCHUNK, _wb, None)


def kernel(x, W):
    tableT = W.transpose(0, 2, 1)
    idxT = x.astype(jnp.int32).T.reshape(_NW, _NCHUNK, _CHUNK)
    outT = _emb_t(tableT, idxT)
    return outT.transpose(2, 0, 1)
